# baseline (speedup 1.0000x reference)
import functools

import jax
import jax.numpy as jnp
from jax import lax
from jax.experimental import pallas as pl
from jax.experimental.pallas import tpu as pltpu
from jax.experimental.pallas import tpu_sc as plsc

N_CORES = 2
N_SUB = 16
NW = N_CORES * N_SUB

N = 10000
E = 320000
D = 128
DE = 16

W = 80
SUP = 2000
GC = 5
GSZ = GC * W
NG = SUP // GSZ
CPS = SUP // W
NSUP = (E // NW) // SUP
N_PAD = 10240
N_PER_SUB = N_PAD // N_SUB
T_PER_SUB = N // N_SUB
ZB = 64


def _pre_body(x_ref, at_ref, we_ref, be_ref, xs_ref, xd_ref, apt_ref):
    xb = x_ref[...]
    we = we_ref[...]
    xs_ref[...] = jnp.dot(xb, we[0:D, :], preferred_element_type=jnp.float32)
    xd_ref[...] = jnp.dot(xb, we[D:2 * D, :], preferred_element_type=jnp.float32)
    apt_ref[...] = lax.dot_general(
        we[2 * D:, :], at_ref[...],
        dimension_numbers=(((0,), (0,)), ((), ())),
        preferred_element_type=jnp.float32,
    ) + be_ref[...]


def _pre(x2d, attr_t, W_e, b_e_col):
    nblk = 1000
    ablk = E // (N // nblk)
    return pl.pallas_call(
        _pre_body,
        grid=(N // nblk,),
        in_specs=[
            pl.BlockSpec((nblk, D), lambda i: (i, 0)),
            pl.BlockSpec((DE, ablk), lambda i: (0, i)),
            pl.BlockSpec((2 * D + DE, DE), lambda i: (0, 0)),
            pl.BlockSpec((DE, 1), lambda i: (0, 0)),
        ],
        out_specs=[
            pl.BlockSpec((nblk, DE), lambda i: (i, 0)),
            pl.BlockSpec((nblk, DE), lambda i: (i, 0)),
            pl.BlockSpec((DE, ablk), lambda i: (0, i)),
        ],
        out_shape=[
            jax.ShapeDtypeStruct((N, DE), jnp.float32),
            jax.ShapeDtypeStruct((N, DE), jnp.float32),
            jax.ShapeDtypeStruct((DE, E), jnp.float32),
        ],
    )(x2d, attr_t, W_e, b_e_col)


def _sc_edge_kernel(xs2d, xd2d, row2, col2, attr_pt):
    mesh = plsc.VectorSubcoreMesh(core_axis_name="c", subcore_axis_name="s")

    @functools.partial(
        pl.kernel,
        out_type=(
            jax.ShapeDtypeStruct((DE, E), jnp.float32),
            jax.ShapeDtypeStruct((N_CORES, N_PAD, DE), jnp.float32),
        ),
        mesh=mesh,
        scratch_types=[
            pltpu.VMEM((CPS, W), jnp.int32),
            pltpu.VMEM((CPS, W), jnp.int32),
            pltpu.VMEM((SUP, DE), jnp.float32),
            pltpu.VMEM((SUP, DE), jnp.float32),
            pltpu.VMEM((2, DE, GSZ), jnp.float32),
            pltpu.VMEM((2, DE, GSZ), jnp.float32),
            pltpu.VMEM((ZB, DE), jnp.float32),
            pltpu.VMEM_SHARED((N_PAD, DE), jnp.float32),
            pltpu.SemaphoreType.DMA,
            pltpu.SemaphoreType.DMA,
            pltpu.SemaphoreType.DMA,
            pltpu.SemaphoreType.DMA,
            pltpu.SemaphoreType.DMA,
            pltpu.SemaphoreType.DMA,
            pltpu.SemaphoreType.DMA,
        ],
        compiler_params=pltpu.CompilerParams(
            use_tc_tiling_on_sc=False, needs_layout_passes=False),
    )
    def k(xs_hbm, xd_hbm, row_hbm, col_hbm, attr_hbm, ea_hbm, agg_hbm,
          row_v, col_v, src_v, dst_v, attr_v, eat_v, zbuf, agg_sp,
          isem, asem0, asem1, gsem0, gsem1, osem0, osem1):
        asem = (asem0, asem1)
        gsem = (gsem0, gsem1)
        osem = (osem0, osem1)
        c = lax.axis_index("c")
        s = lax.axis_index("s")
        wid = s * N_CORES + c
        lane = lax.iota(jnp.int32, DE)

        @pl.loop(0, ZB)
        def _(i):
            zbuf[i, :] = jnp.zeros((DE,), jnp.float32)

        @pl.loop(0, N_PER_SUB // ZB)
        def _(j):
            pltpu.sync_copy(
                zbuf, agg_sp.at[pl.ds(s * N_PER_SUB + j * ZB, ZB)])

        plsc.subcore_barrier()

        @pl.loop(0, NSUP)
        def _(ss):
            eoff = wid * (NSUP * SUP) + ss * SUP
            erow = eoff // W

            i1 = pltpu.async_copy(row_hbm.at[pl.ds(erow, CPS)], row_v, isem)
            i2 = pltpu.async_copy(col_hbm.at[pl.ds(erow, CPS)], col_v, isem)
            i1.wait(); i2.wait()

            def attr_slab(g):
                return attr_hbm.at[pl.ds(0, DE), pl.ds(eoff + g * GSZ, GSZ)]

            def eat_slab(g):
                return ea_hbm.at[pl.ds(0, DE), pl.ds(eoff + g * GSZ, GSZ)]

            def issue(g):
                p = g % 2
                for tt in range(GC):
                    t = g * GC + tt
                    dsl = pl.ds(t * W, W)
                    pltpu.async_copy(
                        xs_hbm.at[row_v.at[t]], src_v.at[dsl], gsem[p])
                    pltpu.async_copy(
                        xd_hbm.at[col_v.at[t]], dst_v.at[dsl], gsem[p])
                pltpu.async_copy(attr_slab(g), attr_v.at[p], asem[p])

            def drain(g):
                p = g % 2
                for tt in range(GC):
                    t = g * GC + tt
                    dsl = pl.ds(t * W, W)
                    pltpu.make_async_copy(
                        xs_hbm.at[row_v.at[t]], src_v.at[dsl],
                        gsem[p]).wait()
                    pltpu.make_async_copy(
                        xd_hbm.at[col_v.at[t]], dst_v.at[dsl],
                        gsem[p]).wait()
                pltpu.make_async_copy(
                    attr_slab(g), attr_v.at[p], asem[p]).wait()

            issue(0)
            for g in range(NG):
                p = g % 2
                if g + 1 < NG:
                    issue(g + 1)
                drain(g)
                if g >= 2:
                    pltpu.make_async_copy(
                        eat_v.at[p], eat_slab(g - 2), osem[p]).wait()

                @pl.loop(0, GSZ)
                def _(i):
                    i2_ = g * GSZ + i
                    icol = jnp.full((DE,), i, jnp.int32)
                    av = plsc.load_gather(attr_v.at[p], [lane, icol])
                    ea = jnp.maximum(src_v[i2_, :] + dst_v[i2_, :] + av, 0.0)
                    src_v[i2_, :] = ea
                    plsc.store_scatter(eat_v.at[p], [lane, icol], ea)

                pltpu.async_copy(eat_v.at[p], eat_slab(g), osem[p])

            pltpu.make_async_copy(
                eat_v.at[(NG - 2) % 2], eat_slab(NG - 2),
                osem[(NG - 2) % 2]).wait()
            pltpu.make_async_copy(
                eat_v.at[(NG - 1) % 2], eat_slab(NG - 1),
                osem[(NG - 1) % 2]).wait()

            @pl.loop(0, CPS)
            def _(t):
                dsl = pl.ds(t * W, W)
                pltpu.sync_copy(
                    src_v.at[dsl], agg_sp.at[col_v.at[t]], add=True)

        plsc.subcore_barrier()
        @pl.loop(0, N_PER_SUB // ZB)
        def _(j):
            nsl = pl.ds(s * N_PER_SUB + j * ZB, ZB)
            pltpu.sync_copy(agg_sp.at[nsl], zbuf)
            pltpu.sync_copy(zbuf, agg_hbm.at[c].at[nsl])

    return k(xs2d, xd2d, row2, col2, attr_pt)


def _node_body(x_ref, agg_ref, wn_ref, bn_ref, out_ref):
    wn = wn_ref[...]
    agg = agg_ref[0] + agg_ref[1]
    out_ref[...] = (
        jnp.dot(x_ref[...], wn[0:D, :], preferred_element_type=jnp.float32)
        + jnp.dot(agg, wn[D:D + DE, :], preferred_element_type=jnp.float32)
        + bn_ref[...]
    )


def _node_update(x2d, agg, W_n, b_n2d):
    blk = 1000
    return pl.pallas_call(
        _node_body,
        grid=(N // blk,),
        in_specs=[
            pl.BlockSpec((blk, D), lambda i: (i, 0)),
            pl.BlockSpec((N_CORES, blk, DE), lambda i: (0, i, 0)),
            pl.BlockSpec((D + DE, D), lambda i: (0, 0)),
            pl.BlockSpec((1, D), lambda i: (0, 0)),
        ],
        out_specs=pl.BlockSpec((blk, D), lambda i: (i, 0)),
        out_shape=jax.ShapeDtypeStruct((N, D), jnp.float32),
    )(x2d, agg, W_n, b_n2d)


def kernel(x, edge_index, edge_attr, W_e, b_e, W_n, b_n):
    x2d = x[0]
    row2 = edge_index[0, 0].reshape(E // W, W)
    col2 = edge_index[0, 1].reshape(E // W, W)
    attr_t = edge_attr[0].T

    xs2d, xd2d, attr_pt = _pre(x2d, attr_t, W_e, b_e.reshape(DE, 1))
    eat, agg = _sc_edge_kernel(xs2d, xd2d, row2, col2, attr_pt)
    x_new = _node_update(x2d, agg, W_n, b_n.reshape(1, D))
    return (x_new[None], eat.T[None])


# speedup vs baseline: 1.1249x; 1.1249x over previous
import functools

import jax
import jax.numpy as jnp
from jax import lax
from jax.experimental import pallas as pl
from jax.experimental.pallas import tpu as pltpu
from jax.experimental.pallas import tpu_sc as plsc

N_CORES = 2
N_SUB = 16
NW = N_CORES * N_SUB

N = 10000
E = 320000
D = 128
DE = 16

W = 80
SUP = 2000
HSUP = SUP // 2
QSUP = SUP // 4
CPS = SUP // W
NSUP = (E // NW) // SUP
N_PAD = 10240
N_PER_SUB = N_PAD // N_SUB
T_PER_SUB = N // N_SUB
ZB = 64


def _pre_body(x_ref, at_ref, we_ref, be_ref, xs_ref, xd_ref, apt_ref):
    xb = x_ref[...]
    we = we_ref[...]
    xs_ref[...] = jnp.dot(xb, we[0:D, :], preferred_element_type=jnp.float32)
    xd_ref[...] = jnp.dot(xb, we[D:2 * D, :], preferred_element_type=jnp.float32)
    apt_ref[...] = lax.dot_general(
        we[2 * D:, :], at_ref[...],
        dimension_numbers=(((0,), (0,)), ((), ())),
        preferred_element_type=jnp.float32,
    ) + be_ref[...]


def _pre(x2d, attr_t, W_e, b_e_col):
    nblk = 2000
    ablk = E // (N // nblk)
    return pl.pallas_call(
        _pre_body,
        grid=(N // nblk,),
        in_specs=[
            pl.BlockSpec((nblk, D), lambda i: (i, 0)),
            pl.BlockSpec((DE, ablk), lambda i: (0, i)),
            pl.BlockSpec((2 * D + DE, DE), lambda i: (0, 0)),
            pl.BlockSpec((DE, 1), lambda i: (0, 0)),
        ],
        out_specs=[
            pl.BlockSpec((nblk, DE), lambda i: (i, 0)),
            pl.BlockSpec((nblk, DE), lambda i: (i, 0)),
            pl.BlockSpec((DE, ablk), lambda i: (0, i)),
        ],
        out_shape=[
            jax.ShapeDtypeStruct((N, DE), jnp.float32),
            jax.ShapeDtypeStruct((N, DE), jnp.float32),
            jax.ShapeDtypeStruct((DE, E), jnp.float32),
        ],
    )(x2d, attr_t, W_e, b_e_col)


def _sc_edge_kernel(xs2d, xd2d, idx2, attr_pt):
    mesh = plsc.VectorSubcoreMesh(core_axis_name="c", subcore_axis_name="s")

    @functools.partial(
        pl.kernel,
        out_type=(
            jax.ShapeDtypeStruct((DE, E), jnp.float32),
            jax.ShapeDtypeStruct((N_CORES, N_PAD, DE), jnp.float32),
        ),
        mesh=mesh,
        scratch_types=[
            pltpu.VMEM((CPS, W), jnp.int32),
            pltpu.VMEM((CPS, W), jnp.int32),
            pltpu.VMEM((SUP, DE), jnp.float32),
            pltpu.VMEM((SUP, DE), jnp.float32),
            pltpu.VMEM((DE, HSUP), jnp.float32),
            pltpu.VMEM((DE, HSUP), jnp.float32),
            pltpu.VMEM((ZB, DE), jnp.float32),
            pltpu.VMEM_SHARED((N_PAD, DE), jnp.float32),
            pltpu.SemaphoreType.DMA,
            pltpu.SemaphoreType.DMA,
            pltpu.SemaphoreType.DMA,
        ],
        compiler_params=pltpu.CompilerParams(
            use_tc_tiling_on_sc=False, needs_layout_passes=False),
    )
    def k(xs_hbm, xd_hbm, idx_hbm, attr_hbm, ea_hbm, agg_hbm,
          row_v, col_v, src_v, dst_v, attr_v, eat_v, zbuf, agg_sp,
          isem, gsem, osem):
        c = lax.axis_index("c")
        s = lax.axis_index("s")
        wid = s * N_CORES + c
        lane = lax.iota(jnp.int32, DE)

        @pl.loop(0, ZB)
        def _(i):
            zbuf[i, :] = jnp.zeros((DE,), jnp.float32)

        @pl.loop(0, N_PER_SUB // ZB)
        def _(j):
            pltpu.sync_copy(
                zbuf, agg_sp.at[pl.ds(s * N_PER_SUB + j * ZB, ZB)])

        plsc.subcore_barrier()

        @pl.loop(0, NSUP)
        def _(ss):
            eoff = wid * (NSUP * SUP) + ss * SUP
            erow = eoff // W

            i1 = pltpu.async_copy(
                idx_hbm.at[0].at[pl.ds(erow, CPS)], row_v, isem)
            i2 = pltpu.async_copy(
                idx_hbm.at[1].at[pl.ds(erow, CPS)], col_v, isem)
            i3 = pltpu.async_copy(
                attr_hbm.at[pl.ds(0, DE), pl.ds(eoff, HSUP)], attr_v, isem)
            i1.wait(); i2.wait(); i3.wait()

            @pl.loop(0, 5)
            def _(g):
                for tt in range(5):
                    t = g * 5 + tt
                    dsl = pl.ds(t * W, W)
                    pltpu.async_copy(
                        xs_hbm.at[row_v.at[t]], src_v.at[dsl], gsem)
                    pltpu.async_copy(
                        xd_hbm.at[col_v.at[t]], dst_v.at[dsl], gsem)
                for tt in range(5):
                    t = g * 5 + tt
                    dsl = pl.ds(t * W, W)
                    pltpu.make_async_copy(
                        xs_hbm.at[row_v.at[t]], src_v.at[dsl], gsem).wait()
                    pltpu.make_async_copy(
                        xd_hbm.at[col_v.at[t]], dst_v.at[dsl], gsem).wait()

            for h in range(2):
                if h == 1:
                    pltpu.sync_copy(
                        attr_hbm.at[pl.ds(0, DE), pl.ds(eoff + HSUP, HSUP)],
                        attr_v)

                @pl.loop(0, HSUP, step=4)
                def _(i):
                    for u in range(4):
                        iu = i + u
                        i2_ = h * HSUP + iu
                        icol = jnp.full((DE,), iu, jnp.int32)
                        av = plsc.load_gather(attr_v, [lane, icol])
                        ea = jnp.maximum(
                            src_v[i2_, :] + dst_v[i2_, :] + av, 0.0)
                        src_v[i2_, :] = ea
                        plsc.store_scatter(eat_v, [lane, icol], ea)

                pltpu.sync_copy(
                    eat_v,
                    ea_hbm.at[pl.ds(0, DE), pl.ds(eoff + h * HSUP, HSUP)])

            @pl.loop(0, CPS)
            def _(t):
                dsl = pl.ds(t * W, W)
                pltpu.sync_copy(
                    src_v.at[dsl], agg_sp.at[col_v.at[t]], add=True)

        plsc.subcore_barrier()
        @pl.loop(0, N_PER_SUB // ZB)
        def _(j):
            nsl = pl.ds(s * N_PER_SUB + j * ZB, ZB)
            pltpu.sync_copy(agg_sp.at[nsl], zbuf)
            pltpu.sync_copy(zbuf, agg_hbm.at[c].at[nsl])

    return k(xs2d, xd2d, idx2, attr_pt)


def _node_body(x_ref, agg_ref, wn_ref, bn_ref, out_ref):
    wn = wn_ref[...]
    agg = agg_ref[0] + agg_ref[1]
    out_ref[...] = (
        jnp.dot(x_ref[...], wn[0:D, :], preferred_element_type=jnp.float32)
        + jnp.dot(agg, wn[D:D + DE, :], preferred_element_type=jnp.float32)
        + bn_ref[...]
    )


def _node_update(x2d, agg, W_n, b_n2d):
    blk = 2000
    return pl.pallas_call(
        _node_body,
        grid=(N // blk,),
        in_specs=[
            pl.BlockSpec((blk, D), lambda i: (i, 0)),
            pl.BlockSpec((N_CORES, blk, DE), lambda i: (0, i, 0)),
            pl.BlockSpec((D + DE, D), lambda i: (0, 0)),
            pl.BlockSpec((1, D), lambda i: (0, 0)),
        ],
        out_specs=pl.BlockSpec((blk, D), lambda i: (i, 0)),
        out_shape=jax.ShapeDtypeStruct((N, D), jnp.float32),
    )(x2d, agg, W_n, b_n2d)


def kernel(x, edge_index, edge_attr, W_e, b_e, W_n, b_n):
    x2d = x[0]
    idx2 = edge_index[0].reshape(2, E // W, W)
    attr_t = edge_attr[0].T

    xs2d, xd2d, attr_pt = _pre(x2d, attr_t, W_e, b_e.reshape(DE, 1))
    eat, agg = _sc_edge_kernel(xs2d, xd2d, idx2, attr_pt)
    x_new = _node_update(x2d, agg, W_n, b_n.reshape(1, D))
    return (x_new[None], eat.T[None])


# speedup vs baseline: 1.1802x; 1.0492x over previous
import functools

import jax
import jax.numpy as jnp
from jax import lax
from jax.experimental import pallas as pl
from jax.experimental.pallas import tpu as pltpu
from jax.experimental.pallas import tpu_sc as plsc

N_CORES = 2
N_SUB = 16
NW = N_CORES * N_SUB

N = 10000
E = 320000
D = 128
DE = 16

W = 80
SUP = 2000
HSUP = SUP // 2
QSUP = SUP // 4
CPS = SUP // W
NSUP = (E // NW) // SUP
N_PAD = 10240
N_PER_SUB = N_PAD // N_SUB
T_PER_SUB = N // N_SUB
ZB = 64


def _pre_body(x_ref, at_ref, we_ref, be_ref, xs_ref, xd_ref, apt_ref):
    xb = x_ref[...]
    we = we_ref[...]
    xs_ref[...] = jnp.dot(xb, we[0:D, :], preferred_element_type=jnp.float32)
    xd_ref[...] = jnp.dot(xb, we[D:2 * D, :], preferred_element_type=jnp.float32)
    apt_ref[...] = lax.dot_general(
        we[2 * D:, :], at_ref[...],
        dimension_numbers=(((0,), (0,)), ((), ())),
        preferred_element_type=jnp.float32,
    ) + be_ref[...]


def _pre(x2d, attr_t, W_e, b_e_col):
    nblk = 2000
    ablk = E // (N // nblk)
    return pl.pallas_call(
        _pre_body,
        grid=(N // nblk,),
        in_specs=[
            pl.BlockSpec((nblk, D), lambda i: (i, 0)),
            pl.BlockSpec((DE, ablk), lambda i: (0, i)),
            pl.BlockSpec((2 * D + DE, DE), lambda i: (0, 0)),
            pl.BlockSpec((DE, 1), lambda i: (0, 0)),
        ],
        out_specs=[
            pl.BlockSpec((nblk, DE), lambda i: (i, 0)),
            pl.BlockSpec((nblk, DE), lambda i: (i, 0)),
            pl.BlockSpec((DE, ablk), lambda i: (0, i)),
        ],
        out_shape=[
            jax.ShapeDtypeStruct((N, DE), jnp.float32),
            jax.ShapeDtypeStruct((N, DE), jnp.float32),
            jax.ShapeDtypeStruct((DE, E), jnp.float32),
        ],
    )(x2d, attr_t, W_e, b_e_col)


def _sc_edge_kernel(xs2d, xd2d, idx2, attr_pt):
    mesh = plsc.VectorSubcoreMesh(core_axis_name="c", subcore_axis_name="s")

    @functools.partial(
        pl.kernel,
        out_type=(
            jax.ShapeDtypeStruct((DE, E), jnp.float32),
            jax.ShapeDtypeStruct((N_CORES, N_PAD, DE), jnp.float32),
        ),
        mesh=mesh,
        scratch_types=[
            pltpu.VMEM((CPS, W), jnp.int32),
            pltpu.VMEM((CPS, W), jnp.int32),
            pltpu.VMEM((SUP, DE), jnp.float32),
            pltpu.VMEM((SUP, DE), jnp.float32),
            pltpu.VMEM((DE, HSUP), jnp.float32),
            pltpu.VMEM((DE, HSUP), jnp.float32),
            pltpu.VMEM((ZB, DE), jnp.float32),
            pltpu.VMEM_SHARED((N_PAD, DE), jnp.float32),
            pltpu.SemaphoreType.DMA,
            pltpu.SemaphoreType.DMA,
            pltpu.SemaphoreType.DMA,
            pltpu.SemaphoreType.DMA,
        ],
        compiler_params=pltpu.CompilerParams(
            use_tc_tiling_on_sc=False, needs_layout_passes=False),
    )
    def k(xs_hbm, xd_hbm, idx_hbm, attr_hbm, ea_hbm, agg_hbm,
          row_v, col_v, src_v, dst_v, attr_v, eat_v, zbuf, agg_sp,
          isem, gsem0, gsem1, osem):
        gsem = (gsem0, gsem1)
        c = lax.axis_index("c")
        s = lax.axis_index("s")
        wid = s * N_CORES + c
        lane = lax.iota(jnp.int32, DE)

        @pl.loop(0, ZB)
        def _(i):
            zbuf[i, :] = jnp.zeros((DE,), jnp.float32)

        @pl.loop(0, N_PER_SUB // ZB)
        def _(j):
            pltpu.sync_copy(
                zbuf, agg_sp.at[pl.ds(s * N_PER_SUB + j * ZB, ZB)])

        plsc.subcore_barrier()

        @pl.loop(0, NSUP)
        def _(ss):
            eoff = wid * (NSUP * SUP) + ss * SUP
            erow = eoff // W

            i1 = pltpu.async_copy(
                idx_hbm.at[0].at[pl.ds(erow, CPS)], row_v, isem)
            i2 = pltpu.async_copy(
                idx_hbm.at[1].at[pl.ds(erow, CPS)], col_v, isem)
            i3 = pltpu.async_copy(
                attr_hbm.at[pl.ds(0, DE), pl.ds(eoff, HSUP)], attr_v, isem)
            i1.wait(); i2.wait(); i3.wait()

            def g_issue(g):
                p = g % 2
                for tt in range(5):
                    t = g * 5 + tt
                    dsl = pl.ds(t * W, W)
                    pltpu.async_copy(
                        xs_hbm.at[row_v.at[t]], src_v.at[dsl], gsem[p])
                    pltpu.async_copy(
                        xd_hbm.at[col_v.at[t]], dst_v.at[dsl], gsem[p])

            def g_drain(g):
                p = g % 2
                for tt in range(5):
                    t = g * 5 + tt
                    dsl = pl.ds(t * W, W)
                    pltpu.make_async_copy(
                        xs_hbm.at[row_v.at[t]], src_v.at[dsl],
                        gsem[p]).wait()
                    pltpu.make_async_copy(
                        xd_hbm.at[col_v.at[t]], dst_v.at[dsl],
                        gsem[p]).wait()

            g_issue(0)
            for g in range(5):
                if g + 1 < 5:
                    g_issue(g + 1)
                g_drain(g)

            for h in range(2):
                if h == 1:
                    pltpu.sync_copy(
                        attr_hbm.at[pl.ds(0, DE), pl.ds(eoff + HSUP, HSUP)],
                        attr_v)

                @pl.loop(0, HSUP, step=4)
                def _(i):
                    for u in range(4):
                        iu = i + u
                        i2_ = h * HSUP + iu
                        icol = jnp.full((DE,), iu, jnp.int32)
                        av = plsc.load_gather(attr_v, [lane, icol])
                        ea = jnp.maximum(
                            src_v[i2_, :] + dst_v[i2_, :] + av, 0.0)
                        src_v[i2_, :] = ea
                        plsc.store_scatter(eat_v, [lane, icol], ea)

                pltpu.sync_copy(
                    eat_v,
                    ea_hbm.at[pl.ds(0, DE), pl.ds(eoff + h * HSUP, HSUP)])

            @pl.loop(0, CPS)
            def _(t):
                dsl = pl.ds(t * W, W)
                pltpu.sync_copy(
                    src_v.at[dsl], agg_sp.at[col_v.at[t]], add=True)

        plsc.subcore_barrier()
        @pl.loop(0, N_PER_SUB // ZB)
        def _(j):
            nsl = pl.ds(s * N_PER_SUB + j * ZB, ZB)
            pltpu.sync_copy(agg_sp.at[nsl], zbuf)
            pltpu.sync_copy(zbuf, agg_hbm.at[c].at[nsl])

    return k(xs2d, xd2d, idx2, attr_pt)


def _node_body(x_ref, agg_ref, wn_ref, bn_ref, out_ref):
    wn = wn_ref[...]
    agg = agg_ref[0] + agg_ref[1]
    out_ref[...] = (
        jnp.dot(x_ref[...], wn[0:D, :], preferred_element_type=jnp.float32)
        + jnp.dot(agg, wn[D:D + DE, :], preferred_element_type=jnp.float32)
        + bn_ref[...]
    )


def _node_update(x2d, agg, W_n, b_n2d):
    blk = 2000
    return pl.pallas_call(
        _node_body,
        grid=(N // blk,),
        in_specs=[
            pl.BlockSpec((blk, D), lambda i: (i, 0)),
            pl.BlockSpec((N_CORES, blk, DE), lambda i: (0, i, 0)),
            pl.BlockSpec((D + DE, D), lambda i: (0, 0)),
            pl.BlockSpec((1, D), lambda i: (0, 0)),
        ],
        out_specs=pl.BlockSpec((blk, D), lambda i: (i, 0)),
        out_shape=jax.ShapeDtypeStruct((N, D), jnp.float32),
    )(x2d, agg, W_n, b_n2d)


def kernel(x, edge_index, edge_attr, W_e, b_e, W_n, b_n):
    x2d = x[0]
    idx2 = edge_index[0].reshape(2, E // W, W)
    attr_t = edge_attr[0].T

    xs2d, xd2d, attr_pt = _pre(x2d, attr_t, W_e, b_e.reshape(DE, 1))
    eat, agg = _sc_edge_kernel(xs2d, xd2d, idx2, attr_pt)
    x_new = _node_update(x2d, agg, W_n, b_n.reshape(1, D))
    return (x_new[None], eat.T[None])
